# baseline (speedup 1.0000x reference)
import functools

import jax
import jax.numpy as jnp
from jax import lax
from jax.experimental import pallas as pl
from jax.experimental.pallas import tpu as pltpu
from jax.experimental.pallas import tpu_sc as plsc

_ROW_BLK = 4000
_CHUNK = 128


def _fuse_body(aw_ref, adj_ref, attrW_ref, itemW_ref, out_ref):
    acc = jnp.dot(adj_ref[...], attrW_ref[...], preferred_element_type=jnp.float32)
    hidden = acc.shape[1]
    out_ref[:, pl.ds(0, hidden)] = itemW_ref[...] + aw_ref[0] * acc


def _fused_table(attr_weight, adj, attr_W, item_W):
    rows, att = adj.shape
    hidden = attr_W.shape[1]
    grid = (rows // _ROW_BLK,)
    return pl.pallas_call(
        _fuse_body,
        grid=grid,
        in_specs=[
            pl.BlockSpec(memory_space=pltpu.SMEM),
            pl.BlockSpec((_ROW_BLK, att), lambda i: (i, 0)),
            pl.BlockSpec((att, hidden), lambda i: (0, 0)),
            pl.BlockSpec((_ROW_BLK, hidden), lambda i: (i, 0)),
        ],
        out_specs=pl.BlockSpec((_ROW_BLK, 2 * hidden), lambda i: (i, 0)),
        out_shape=jax.ShapeDtypeStruct((rows, 2 * hidden), jnp.float32),
    )(attr_weight, adj, attr_W, item_W)


@functools.lru_cache(maxsize=None)
def _make_gather(b, s, hidden):
    info = plsc.get_sparse_core_info()
    nc, ns = info.num_cores, info.num_subcores
    nw = nc * ns
    assert b % (nw * _CHUNK) == 0 or b == nw * _CHUNK
    assert hidden % 8 == 0
    hh_n = hidden // 8
    mesh = plsc.VectorSubcoreMesh(core_axis_name="c", subcore_axis_name="s")

    nbuf = 8
    assert s % 2 == 0 and (s - 2) % nbuf == 0

    @functools.partial(
        pl.kernel,
        mesh=mesh,
        compiler_params=pltpu.CompilerParams(
            use_tc_tiling_on_sc=False, needs_layout_passes=False),
        out_type=jax.ShapeDtypeStruct((s, hh_n, nw, 8, _CHUNK), jnp.float32),
        scratch_types=[
            pltpu.VMEM((s, _CHUNK), jnp.int32),
            pltpu.VMEM((nbuf, _CHUNK, hidden), jnp.float32),
            pltpu.VMEM((2, hidden // 8, 8, _CHUNK + 1), jnp.float32),
            [pltpu.SemaphoreType.DMA] * nbuf,
            [pltpu.SemaphoreType.DMA] * 2,
        ],
    )
    def gather_k(table_hbm, idx_hbm, out_hbm, idx_all, rows_v, tr_v, gsem, wsem):
        wid = lax.axis_index("s") * nc + lax.axis_index("c")
        iota = lax.iota(jnp.int32, 16)
        hh_ids = [(16 * k + iota) // 8 for k in range(hidden // 16)]
        hi_ids = iota % 8
        last = jnp.int32(s - 1)

        def fire_gather(si, u):
            si = jnp.minimum(si, last)
            pltpu.async_copy(table_hbm.at[idx_all.at[si]],
                             rows_v.at[u], gsem[u])

        def wait_gather(si, u):
            si = jnp.minimum(si, last)
            pltpu.make_async_copy(table_hbm.at[idx_all.at[si]],
                                  rows_v.at[u], gsem[u]).wait()

        def fire_writes(si, tp):
            pltpu.async_copy(
                tr_v.at[tp, :, :, pl.ds(0, _CHUNK)],
                out_hbm.at[si, :, wid], wsem[tp])

        def wait_writes(si, tp):
            pltpu.make_async_copy(
                tr_v.at[tp, :, :, pl.ds(0, _CHUNK)],
                out_hbm.at[si, :, wid], wsem[tp]).wait()

        def transpose_chunk(u, tp):
            rows_ref = rows_v.at[u]
            tr_ref = tr_v.at[tp]

            def tbody(jj, carry):
                for v8 in range(8):
                    bi = 8 * jj + v8
                    col = jnp.full((16,), bi, dtype=jnp.int32)
                    for k in range(hidden // 16):
                        v = rows_ref[bi, pl.ds(16 * k, 16)]
                        plsc.store_scatter(
                            tr_ref, [hh_ids[k], hi_ids, col], v)
                return carry

            lax.fori_loop(0, _CHUNK // 8, tbody, 0)

        def step(si, u, tp, first):
            wait_gather(si, u)
            if not first:
                wait_writes(si - 2, tp)
            transpose_chunk(u, tp)
            fire_gather(si + nbuf, u)
            fire_writes(si, tp)

        pltpu.sync_copy(idx_hbm.at[:, pl.ds(wid * _CHUNK, _CHUNK)], idx_all)
        for u in range(nbuf):
            fire_gather(jnp.int32(u), u)

        def body(t, carry):
            for u in range(nbuf):
                si = nbuf * t + u
                step(si, u, u % 2, False)
            return carry

        for u in range(nbuf):
            step(jnp.int32(u), u, u % 2, u < 2)
        lax.fori_loop(1, (s - 2) // nbuf, body, 0)
        step(jnp.int32(s - 2), (s - 2) % nbuf, 0, False)
        step(jnp.int32(s - 1), (s - 1) % nbuf, 1, False)
        for u in range(nbuf):
            wait_gather(jnp.int32(s - 1), u)
        wait_writes(jnp.int32(s - 2), 0)
        wait_writes(jnp.int32(s - 1), 1)

    return gather_k


def kernel(item_list, attr_weight, adj, attr_W, item_W):
    b, s = item_list.shape
    rows, hidden = item_W.shape
    fused = _fused_table(attr_weight, adj, attr_W, item_W).reshape(2 * rows, hidden)
    idx = item_list.T.astype(jnp.int32) * 2
    out5 = _make_gather(b, s, hidden)(fused, idx)
    return out5.transpose((2, 4, 0, 1, 3)).reshape(b, s, hidden)


# speedup vs baseline: 1.0070x; 1.0070x over previous
import functools

import jax
import jax.numpy as jnp
from jax import lax
from jax.experimental import pallas as pl
from jax.experimental.pallas import tpu as pltpu
from jax.experimental.pallas import tpu_sc as plsc

_ROW_BLK = 4000
_CHUNK = 128


def _fuse_body(aw_ref, adj_ref, attrW_ref, itemW_ref, out_ref):
    acc = jnp.dot(adj_ref[...], attrW_ref[...], preferred_element_type=jnp.float32)
    hidden = acc.shape[1]
    out_ref[:, pl.ds(0, hidden)] = itemW_ref[...] + aw_ref[0] * acc


def _fused_table(attr_weight, adj, attr_W, item_W):
    rows, att = adj.shape
    hidden = attr_W.shape[1]
    grid = (rows // _ROW_BLK,)
    return pl.pallas_call(
        _fuse_body,
        grid=grid,
        in_specs=[
            pl.BlockSpec(memory_space=pltpu.SMEM),
            pl.BlockSpec((_ROW_BLK, att), lambda i: (i, 0)),
            pl.BlockSpec((att, hidden), lambda i: (0, 0)),
            pl.BlockSpec((_ROW_BLK, hidden), lambda i: (i, 0)),
        ],
        out_specs=pl.BlockSpec((_ROW_BLK, 2 * hidden), lambda i: (i, 0)),
        out_shape=jax.ShapeDtypeStruct((rows, 2 * hidden), jnp.float32),
    )(attr_weight, adj, attr_W, item_W)


@functools.lru_cache(maxsize=None)
def _make_gather(b, s, hidden):
    info = plsc.get_sparse_core_info()
    nc, ns = info.num_cores, info.num_subcores
    nw = nc * ns
    assert b % (nw * _CHUNK) == 0 or b == nw * _CHUNK
    assert hidden % 8 == 0
    hh_n = hidden // 8
    mesh = plsc.VectorSubcoreMesh(core_axis_name="c", subcore_axis_name="s")

    nbuf = 6
    assert s % 2 == 0 and (s - 2) % nbuf == 0

    @functools.partial(
        pl.kernel,
        mesh=mesh,
        compiler_params=pltpu.CompilerParams(
            use_tc_tiling_on_sc=False, needs_layout_passes=False),
        out_type=jax.ShapeDtypeStruct((s, hh_n, nw, 8, _CHUNK), jnp.float32),
        scratch_types=[
            pltpu.VMEM((s, _CHUNK), jnp.int32),
            pltpu.VMEM((nbuf, _CHUNK, hidden), jnp.float32),
            pltpu.VMEM((2, hidden // 8, 8, _CHUNK + 1), jnp.float32),
            [pltpu.SemaphoreType.DMA] * nbuf,
            [pltpu.SemaphoreType.DMA] * 2,
        ],
    )
    def gather_k(table_hbm, idx_hbm, out_hbm, idx_all, rows_v, tr_v, gsem, wsem):
        wid = lax.axis_index("s") * nc + lax.axis_index("c")
        iota = lax.iota(jnp.int32, 16)
        hh_ids = [(16 * k + iota) // 8 for k in range(hidden // 16)]
        hi_ids = iota % 8
        last = jnp.int32(s - 1)

        def fire_gather(si, u):
            si = jnp.minimum(si, last)
            pltpu.async_copy(table_hbm.at[idx_all.at[si]],
                             rows_v.at[u], gsem[u])

        def wait_gather(si, u):
            si = jnp.minimum(si, last)
            pltpu.make_async_copy(table_hbm.at[idx_all.at[si]],
                                  rows_v.at[u], gsem[u]).wait()

        def fire_writes(si, tp):
            pltpu.async_copy(
                tr_v.at[tp, :, :, pl.ds(0, _CHUNK)],
                out_hbm.at[si, :, wid], wsem[tp])

        def wait_writes(si, tp):
            pltpu.make_async_copy(
                tr_v.at[tp, :, :, pl.ds(0, _CHUNK)],
                out_hbm.at[si, :, wid], wsem[tp]).wait()

        def transpose_chunk(u, tp):
            rows_ref = rows_v.at[u]
            tr_ref = tr_v.at[tp]

            def tbody(jj, carry):
                for v8 in range(8):
                    bi = 8 * jj + v8
                    col = jnp.full((16,), bi, dtype=jnp.int32)
                    for k in range(hidden // 16):
                        v = rows_ref[bi, pl.ds(16 * k, 16)]
                        plsc.store_scatter(
                            tr_ref, [hh_ids[k], hi_ids, col], v)
                return carry

            lax.fori_loop(0, _CHUNK // 8, tbody, 0)

        def step(si, u, tp, first):
            wait_gather(si, u)
            if not first:
                wait_writes(si - 2, tp)
            transpose_chunk(u, tp)
            fire_gather(si + nbuf, u)
            fire_writes(si, tp)

        pltpu.sync_copy(idx_hbm.at[:, pl.ds(wid * _CHUNK, _CHUNK)], idx_all)
        for u in range(nbuf):
            fire_gather(jnp.int32(u), u)

        def body(t, carry):
            for u in range(nbuf):
                si = nbuf * t + u
                step(si, u, u % 2, False)
            return carry

        for u in range(nbuf):
            step(jnp.int32(u), u, u % 2, u < 2)
        lax.fori_loop(1, (s - 2) // nbuf, body, 0)
        step(jnp.int32(s - 2), (s - 2) % nbuf, 0, False)
        step(jnp.int32(s - 1), (s - 1) % nbuf, 1, False)
        for u in range(nbuf):
            wait_gather(jnp.int32(s - 1), u)
        wait_writes(jnp.int32(s - 2), 0)
        wait_writes(jnp.int32(s - 1), 1)

    return gather_k


def kernel(item_list, attr_weight, adj, attr_W, item_W):
    b, s = item_list.shape
    rows, hidden = item_W.shape
    fused = _fused_table(attr_weight, adj, attr_W, item_W).reshape(2 * rows, hidden)
    idx = item_list.T.astype(jnp.int32) * 2
    out5 = _make_gather(b, s, hidden)(fused, idx)
    return out5.transpose((2, 4, 0, 1, 3)).reshape(b, s, hidden)
